# baseline (speedup 1.0000x reference)
import functools

import jax
import jax.numpy as jnp
from jax import lax
from jax.experimental import pallas as pl
from jax.experimental.pallas import tpu as pltpu
from jax.experimental.pallas import tpu_sc as plsc

N = 10000
E = 320000
D = 128
NC = 2
NS = 16
DH = D // NC
C = 80
NB = 5
TILE_EDGES = E // NS
CHUNKS = TILE_EDGES // C
NPAD = 10240
ROWS_PER_TILE = NPAD // NS
ZC = 80
DEGW = 8
BR = 1024
EP = E // 128


def _make_sc_body(with_deg):
  def body(*refs):
    if with_deg:
        (hw_hbm, srca_hbm, dst_hbm, p3_hbm, deg_hbm,
         src_idx, dst_idx, rows, zbuf, ones_v, dstage,
         acc, dacc, gsem, ssem, isem, dsem) = refs
    else:
        (hw_hbm, srca_hbm, dst_hbm, degin_hbm, p3_hbm,
         src_idx, dst_idx, rows, zbuf, dstage,
         acc, gsem, ssem, isem) = refs
    cid = lax.axis_index("c")
    sid = lax.axis_index("s")

    pltpu.async_copy(srca_hbm.at[sid], src_idx, isem)
    pltpu.async_copy(dst_hbm.at[sid], dst_idx, isem)
    base = sid * ROWS_PER_TILE
    if not with_deg:
        pltpu.async_copy(degin_hbm.at[pl.ds(base, ROWS_PER_TILE)], dstage, isem)

    @pl.loop(0, ZC)
    def _fillz(i):
        for j in range(DH // 16):
            zbuf[i, pl.ds(j * 16, 16)] = jnp.zeros((16,), jnp.float32)

    if with_deg:
        @pl.loop(0, C)
        def _fill1(i):
            ones_v[i, :] = jnp.ones((DEGW,), jnp.float32)

        @pl.loop(0, ROWS_PER_TILE)
        def _fill0(i):
            dstage[i, :] = jnp.zeros((DEGW,), jnp.float32)

    @pl.loop(0, ROWS_PER_TILE // ZC)
    def _zero(k):
        pltpu.sync_copy(zbuf, acc.at[pl.ds(base + k * ZC, ZC)])
    if with_deg:
        pltpu.sync_copy(dstage, dacc.at[pl.ds(base, ROWS_PER_TILE)])

    pltpu.make_async_copy(dst_hbm.at[sid], src_idx, isem).wait()
    pltpu.make_async_copy(dst_hbm.at[sid], dst_idx, isem).wait()

    @pl.when(cid == 1)
    def _bump():
        @pl.loop(0, CHUNKS)
        def _row(j):
            for g in range(C // 16):
                sl = pl.ds(g * 16, 16)
                src_idx[j, sl] = src_idx[j, sl] + 1

    plsc.subcore_barrier()

    table = hw_hbm.at[0]

    def start_gather(j, b):
        pltpu.async_copy(table.at[src_idx.at[j]], rows.at[b], gsem.at[b])

    def wait_gather(b):
        pltpu.make_async_copy(table.at[src_idx.at[0]], rows.at[b],
                              gsem.at[b]).wait()

    def start_scatter(j, b):
        pltpu.async_copy(rows.at[b], acc.at[dst_idx.at[j]], ssem.at[b],
                         add=True)
        if with_deg:
            pltpu.async_copy(ones_v, dacc.at[dst_idx.at[j]], dsem.at[b],
                             add=True)

    def wait_scatter(b):
        pltpu.make_async_copy(rows.at[b], acc.at[dst_idx.at[0]],
                              ssem.at[b]).wait()
        if with_deg:
            pltpu.make_async_copy(ones_v, dacc.at[dst_idx.at[0]],
                                  dsem.at[b]).wait()

    for b in range(NB):
        start_gather(b, b)

    @pl.loop(0, CHUNKS - NB, step=NB)
    def _main(j):
        for b in range(NB):
            wait_gather(b)
            start_scatter(j + b, b)
        for b in range(NB):
            wait_scatter(b)
            start_gather(j + NB + b, b)

    for b in range(NB):
        wait_gather(b)
        start_scatter(CHUNKS - NB + b, b)
    for b in range(NB):
        wait_scatter(b)

    plsc.subcore_barrier()
    if with_deg:
        pltpu.sync_copy(dacc.at[pl.ds(base, ROWS_PER_TILE)], dstage)

        @pl.when(cid == 0)
        def _deg_out():
            pltpu.sync_copy(dstage, deg_hbm.at[pl.ds(base, ROWS_PER_TILE)])

    @pl.loop(0, ROWS_PER_TILE // ZC)
    def _wb(k):
        pltpu.sync_copy(acc.at[pl.ds(base + k * ZC, ZC)], zbuf)

        @pl.loop(0, ZC)
        def _scale(i):
            d = jnp.maximum(dstage[k * ZC + i, :][0], 1.0)
            for g in range(DH // 16):
                sl = pl.ds(g * 16, 16)
                zbuf[i, sl] = zbuf[i, sl] / d

        pltpu.sync_copy(zbuf, p3_hbm.at[pl.ds(base + k * ZC, ZC), cid])

  return body


@functools.cache
def _get_sc_agg(with_deg):
  out_type = [jax.ShapeDtypeStruct((NPAD, NC, DH), jnp.float32)]
  scratch = [
      pltpu.VMEM((CHUNKS, C), jnp.int32),
      pltpu.VMEM((CHUNKS, C), jnp.int32),
      pltpu.VMEM((NB, C, DH), jnp.float32),
      pltpu.VMEM((ZC, DH), jnp.float32),
  ]
  if with_deg:
      out_type.append(jax.ShapeDtypeStruct((NPAD, DEGW), jnp.float32))
      scratch.append(pltpu.VMEM((C, DEGW), jnp.float32))
  scratch.append(pltpu.VMEM((ROWS_PER_TILE, DEGW), jnp.float32))
  scratch.append(pltpu.VMEM_SHARED((NPAD, DH), jnp.float32))
  if with_deg:
      scratch.append(pltpu.VMEM_SHARED((NPAD, DEGW), jnp.float32))
  scratch.append(pltpu.SemaphoreType.DMA((NB,)))
  scratch.append(pltpu.SemaphoreType.DMA((NB,)))
  scratch.append(pltpu.SemaphoreType.DMA)
  if with_deg:
      scratch.append(pltpu.SemaphoreType.DMA((NB,)))
  return pl.kernel(
      _make_sc_body(with_deg),
      out_type=tuple(out_type) if with_deg else out_type[0],
      mesh=plsc.VectorSubcoreMesh(core_axis_name="c", subcore_axis_name="s",
                                  num_cores=NC, num_subcores=NS),
      compiler_params=pltpu.CompilerParams(use_tc_tiling_on_sc=False),
      scratch_types=scratch,
  )


def _dot_t(a, w):
    return lax.dot_general(a, w, (((1,), (1,)), ((), ())),
                           preferred_element_type=jnp.float32)


def _emb_body(x_ref, wemb_ref, bemb_ref, wl_ref, ei_ref,
              h_ref, hw_ref, srca_ref, dst_ref):
    h = _dot_t(x_ref[...], wemb_ref[...]) + bemb_ref[...]
    h_ref[...] = h
    hw_ref[...] = _dot_t(h, wl_ref[...])

    @pl.when(pl.program_id(0) == 0)
    def _idx():
        srca_ref[...] = (ei_ref[0] * 2).reshape(EP, 128)
        dst_ref[...] = ei_ref[1].reshape(EP, 128)


_emb = pl.pallas_call(
    _emb_body,
    grid=(NPAD // BR,),
    in_specs=[pl.BlockSpec((BR, D), lambda i: (i, 0)),
              pl.BlockSpec((D, D), lambda i: (0, 0)),
              pl.BlockSpec((1, D), lambda i: (0, 0)),
              pl.BlockSpec((D, D), lambda i: (0, 0)),
              pl.BlockSpec((2, E), lambda i: (0, 0))],
    out_specs=[pl.BlockSpec((BR, D), lambda i: (i, 0)),
               pl.BlockSpec((BR, D), lambda i: (i, 0)),
               pl.BlockSpec((EP, 128), lambda i: (0, 0)),
               pl.BlockSpec((EP, 128), lambda i: (0, 0))],
    out_shape=[jax.ShapeDtypeStruct((NPAD, D), jnp.float32),
               jax.ShapeDtypeStruct((NPAD, D), jnp.float32),
               jax.ShapeDtypeStruct((EP, 128), jnp.int32),
               jax.ShapeDtypeStruct((EP, 128), jnp.int32)],
)


def _make_combine(with_relu, with_next):
    def body(*refs):
        if with_next:
            p_ref, h_ref, wr_ref, bl_ref, wl_ref, out_ref, hw_ref = refs
        else:
            p_ref, h_ref, wr_ref, bl_ref, out_ref = refs
        t = p_ref[...] + _dot_t(h_ref[...], wr_ref[...]) + bl_ref[...]
        if with_relu:
            t = jnp.maximum(t, 0.0)
        out_ref[...] = t
        if with_next:
            hw_ref[...] = _dot_t(t, wl_ref[...])

    in_specs = [pl.BlockSpec((BR, D), lambda i: (i, 0)),
                pl.BlockSpec((BR, D), lambda i: (i, 0)),
                pl.BlockSpec((D, D), lambda i: (0, 0)),
                pl.BlockSpec((1, D), lambda i: (0, 0))]
    out_specs = [pl.BlockSpec((BR, D), lambda i: (i, 0))]
    out_shape = [jax.ShapeDtypeStruct((NPAD, D), jnp.float32)]
    if with_next:
        in_specs.append(pl.BlockSpec((D, D), lambda i: (0, 0)))
        out_specs = out_specs + [pl.BlockSpec((BR, D), lambda i: (i, 0))]
        out_shape = out_shape + [jax.ShapeDtypeStruct((NPAD, D), jnp.float32)]
    return pl.pallas_call(
        body, grid=(NPAD // BR,),
        in_specs=in_specs, out_specs=out_specs, out_shape=out_shape)


_combine_next = _make_combine(True, True)
_combine_last = _make_combine(False, False)


def kernel(x, edge_index, W_emb, b_emb, Wl0, bl0, Wr0, Wl1, bl1, Wr1, Wl2, bl2, Wr2):
    b_emb2 = b_emb.reshape(1, D)
    bl0_2 = bl0.reshape(1, D)
    bl1_2 = bl1.reshape(1, D)
    bl2_2 = bl2.reshape(1, D)

    sc_deg = _get_sc_agg(True)
    sc_plain = _get_sc_agg(False)

    xp = jnp.pad(x, ((0, NPAD - N), (0, 0)))
    h0, hw, srca_p, dst_p = _emb(xp, W_emb, b_emb2, Wl0, edge_index)
    srca3 = srca_p.reshape(NS, CHUNKS, C)
    dst3 = dst_p.reshape(NS, CHUNKS, C)

    def half_view(hwfull):
        return hwfull.reshape(1, NC * NPAD, DH)

    p3, deg16 = sc_deg(half_view(hw), srca3, dst3)
    h1, hw = _combine_next(p3.reshape(NPAD, D), h0, Wr0, bl0_2, Wl1)
    p3 = sc_plain(half_view(hw), srca3, dst3, deg16)
    h2, hw = _combine_next(p3.reshape(NPAD, D), h1, Wr1, bl1_2, Wl2)
    p3 = sc_plain(half_view(hw), srca3, dst3, deg16)
    out, = _combine_last(p3.reshape(NPAD, D), h2, Wr2, bl2_2)
    return out[:N]


# speedup vs baseline: 1.1661x; 1.1661x over previous
import functools

import jax
import jax.numpy as jnp
from jax import lax
from jax.experimental import pallas as pl
from jax.experimental.pallas import tpu as pltpu
from jax.experimental.pallas import tpu_sc as plsc

N = 10000
E = 320000
D = 128
NC = 2
NS = 16
DH = D // NC
C = 100
NB = 4
TILE_EDGES = E // NS
CHUNKS = TILE_EDGES // C
NPAD = 10240
ROWS_PER_TILE = NPAD // NS
ZC = 80
DEGW = 16
BR = 1000


def _make_sc_body(with_deg):
  def body(*refs):
    if with_deg:
        (hw_hbm, src_hbm, dst_hbm, p_hbm, deg_hbm,
         src_idx, dst_idx, rows, zbuf, ones_v, zeros_v,
         acc, dacc, gsem, ssem, isem, dsem) = refs
    else:
        (hw_hbm, src_hbm, dst_hbm, p_hbm,
         src_idx, dst_idx, rows, zbuf,
         acc, gsem, ssem, isem) = refs
    cid = lax.axis_index("c")
    sid = lax.axis_index("s")

    pltpu.async_copy(src_hbm.at[sid], src_idx, isem)
    pltpu.async_copy(dst_hbm.at[sid], dst_idx, isem)

    @pl.loop(0, ZC)
    def _fillz(i):
        for j in range(DH // 16):
            zbuf[i, pl.ds(j * 16, 16)] = jnp.zeros((16,), jnp.float32)

    if with_deg:
        @pl.loop(0, C)
        def _fill1(i):
            ones_v[i, :] = jnp.ones((DEGW,), jnp.float32)

        @pl.loop(0, ZC)
        def _fill0(i):
            zeros_v[i, :] = jnp.zeros((DEGW,), jnp.float32)

    base = sid * ROWS_PER_TILE

    @pl.loop(0, ROWS_PER_TILE // ZC)
    def _zero(k):
        pltpu.sync_copy(zbuf, acc.at[pl.ds(base + k * ZC, ZC)])
        if with_deg:
            pltpu.sync_copy(zeros_v, dacc.at[pl.ds(base + k * ZC, ZC)])

    pltpu.make_async_copy(src_hbm.at[sid], src_idx, isem).wait()
    pltpu.make_async_copy(dst_hbm.at[sid], dst_idx, isem).wait()

    plsc.subcore_barrier()

    hw_half = hw_hbm.at[cid]

    def start_gather(j, b):
        pltpu.async_copy(hw_half.at[src_idx.at[j]], rows.at[b], gsem.at[b])

    def wait_gather(b):
        pltpu.make_async_copy(hw_half.at[src_idx.at[0]], rows.at[b],
                              gsem.at[b]).wait()

    def start_scatter(j, b):
        pltpu.async_copy(rows.at[b], acc.at[dst_idx.at[j]], ssem.at[b],
                         add=True)
        if with_deg:
            @pl.when(cid == 0)
            def _deg():
                pltpu.async_copy(ones_v, dacc.at[dst_idx.at[j]], dsem.at[b],
                                 add=True)

    def wait_scatter(b):
        pltpu.make_async_copy(rows.at[b], acc.at[dst_idx.at[0]],
                              ssem.at[b]).wait()
        if with_deg:
            @pl.when(cid == 0)
            def _deg():
                pltpu.make_async_copy(ones_v, dacc.at[dst_idx.at[0]],
                                      dsem.at[b]).wait()

    for b in range(NB):
        start_gather(b, b)

    @pl.loop(0, CHUNKS - NB, step=NB)
    def _main(j):
        for b in range(NB):
            wait_gather(b)
            start_scatter(j + b, b)
        for b in range(NB):
            wait_scatter(b)
            start_gather(j + NB + b, b)

    for b in range(NB):
        wait_gather(b)
        start_scatter(CHUNKS - NB + b, b)
    for b in range(NB):
        wait_scatter(b)

    plsc.subcore_barrier()
    pltpu.sync_copy(acc.at[pl.ds(base, ROWS_PER_TILE)],
                    p_hbm.at[cid, pl.ds(base, ROWS_PER_TILE)])

    if with_deg:
        @pl.when(cid == 0)
        def _deg_out():
            pltpu.sync_copy(dacc.at[pl.ds(base, ROWS_PER_TILE)],
                            deg_hbm.at[pl.ds(base, ROWS_PER_TILE)])

  return body


@functools.cache
def _get_sc_agg(with_deg):
  out_type = [jax.ShapeDtypeStruct((NC, NPAD, DH), jnp.float32)]
  scratch = [
      pltpu.VMEM((CHUNKS, C), jnp.int32),
      pltpu.VMEM((CHUNKS, C), jnp.int32),
      pltpu.VMEM((NB, C, DH), jnp.float32),
      pltpu.VMEM((ZC, DH), jnp.float32),
  ]
  if with_deg:
      out_type.append(jax.ShapeDtypeStruct((NPAD, DEGW), jnp.float32))
      scratch.append(pltpu.VMEM((C, DEGW), jnp.float32))
      scratch.append(pltpu.VMEM((ZC, DEGW), jnp.float32))
  scratch.append(pltpu.VMEM_SHARED((NPAD, DH), jnp.float32))
  if with_deg:
      scratch.append(pltpu.VMEM_SHARED((NPAD, DEGW), jnp.float32))
  scratch.append(pltpu.SemaphoreType.DMA((NB,)))
  scratch.append(pltpu.SemaphoreType.DMA((NB,)))
  scratch.append(pltpu.SemaphoreType.DMA)
  if with_deg:
      scratch.append(pltpu.SemaphoreType.DMA((NB,)))
  return pl.kernel(
      _make_sc_body(with_deg),
      out_type=tuple(out_type) if with_deg else out_type[0],
      mesh=plsc.VectorSubcoreMesh(core_axis_name="c", subcore_axis_name="s",
                                  num_cores=NC, num_subcores=NS),
      compiler_params=pltpu.CompilerParams(use_tc_tiling_on_sc=False),
      scratch_types=scratch,
  )


def _dot_t(a, w):
    return lax.dot_general(a, w, (((1,), (1,)), ((), ())),
                           preferred_element_type=jnp.float32)


def _split_store(hw_ref, t):
    hw_ref[0] = t[:, :DH]
    hw_ref[1] = t[:, DH:]


def _emb_body(x_ref, wemb_ref, bemb_ref, wl_ref, h_ref, hw_ref):
    h = _dot_t(x_ref[...], wemb_ref[...]) + bemb_ref[...]
    h_ref[...] = h
    _split_store(hw_ref, _dot_t(h, wl_ref[...]))


_emb = pl.pallas_call(
    _emb_body,
    grid=(N // BR,),
    in_specs=[pl.BlockSpec((BR, D), lambda i: (i, 0)),
              pl.BlockSpec((D, D), lambda i: (0, 0)),
              pl.BlockSpec((1, D), lambda i: (0, 0)),
              pl.BlockSpec((D, D), lambda i: (0, 0))],
    out_specs=[pl.BlockSpec((BR, D), lambda i: (i, 0)),
               pl.BlockSpec((NC, BR, DH), lambda i: (0, i, 0))],
    out_shape=[jax.ShapeDtypeStruct((N, D), jnp.float32),
               jax.ShapeDtypeStruct((NC, N, DH), jnp.float32)],
)


def _make_combine(with_relu, with_next):
    def body(*refs):
        if with_next:
            (p_ref, d_ref, h_ref, wr_ref, bl_ref, wl_ref,
             out_ref, hw_ref) = refs
        else:
            p_ref, d_ref, h_ref, wr_ref, bl_ref, out_ref = refs
        deg = d_ref[:, 0]
        scale = 1.0 / jnp.maximum(deg, 1.0)
        agg = jnp.concatenate([p_ref[0], p_ref[1]], axis=1)
        t = agg * scale[:, None]
        t = t + _dot_t(h_ref[...], wr_ref[...]) + bl_ref[...]
        if with_relu:
            t = jnp.maximum(t, 0.0)
        out_ref[...] = t
        if with_next:
            _split_store(hw_ref, _dot_t(t, wl_ref[...]))

    in_specs = [pl.BlockSpec((NC, BR, DH), lambda i: (0, i, 0)),
                pl.BlockSpec((BR, DEGW), lambda i: (i, 0)),
                pl.BlockSpec((BR, D), lambda i: (i, 0)),
                pl.BlockSpec((D, D), lambda i: (0, 0)),
                pl.BlockSpec((1, D), lambda i: (0, 0))]
    out_specs = [pl.BlockSpec((BR, D), lambda i: (i, 0))]
    out_shape = [jax.ShapeDtypeStruct((N, D), jnp.float32)]
    if with_next:
        in_specs.append(pl.BlockSpec((D, D), lambda i: (0, 0)))
        out_specs = out_specs + [pl.BlockSpec((NC, BR, DH), lambda i: (0, i, 0))]
        out_shape = out_shape + [jax.ShapeDtypeStruct((NC, N, DH), jnp.float32)]
    return pl.pallas_call(
        body, grid=(N // BR,),
        in_specs=in_specs, out_specs=out_specs, out_shape=out_shape)


_combine_next = _make_combine(True, True)
_combine_last = _make_combine(False, False)


def kernel(x, edge_index, W_emb, b_emb, Wl0, bl0, Wr0, Wl1, bl1, Wr1, Wl2, bl2, Wr2):
    src2 = edge_index[0].reshape(NS, CHUNKS, C)
    dst2 = edge_index[1].reshape(NS, CHUNKS, C)
    b_emb2 = b_emb.reshape(1, D)
    bl0_2 = bl0.reshape(1, D)
    bl1_2 = bl1.reshape(1, D)
    bl2_2 = bl2.reshape(1, D)

    sc_deg = _get_sc_agg(True)
    sc_plain = _get_sc_agg(False)
    h0, hw0 = _emb(x, W_emb, b_emb2, Wl0)
    p, degp = sc_deg(hw0, src2, dst2)
    h1, hw1 = _combine_next(p, degp, h0, Wr0, bl0_2, Wl1)
    p = sc_plain(hw1, src2, dst2)
    h2, hw2 = _combine_next(p, degp, h1, Wr1, bl1_2, Wl2)
    p = sc_plain(hw2, src2, dst2)
    out, = _combine_last(p, degp, h2, Wr2, bl2_2)
    return out
